# baseline (speedup 1.0000x reference)
import functools

import jax
import jax.numpy as jnp
from jax import lax
from jax.experimental import pallas as pl
from jax.experimental.pallas import tpu as pltpu
from jax.experimental.pallas import tpu_sc as plsc

N_EDGES = 320000
NUM_RELS = 1000
D = 128
LANES = 16
SUBVECS = D // LANES
NUM_CORES = 2
NUM_SUBCORES = 16
NUM_WORKERS = NUM_CORES * NUM_SUBCORES
CHUNK = 128
TOTAL_CHUNKS = N_EDGES // CHUNK
CHUNKS_LO = TOTAL_CHUNKS // NUM_WORKERS
CHUNKS_HI = CHUNKS_LO + 1
NUM_HI = TOTAL_CHUNKS - CHUNKS_LO * NUM_WORKERS
GROUPS = CHUNK // LANES


def _sc_kernel(x_i_hbm, x_j_hbm, et_hbm, table_hbm, out_hbm,
               table_sh, xa_v, xj_v, idx_v, out_v, part_v,
               sem_tab, sem_xa, sem_ga, sem_xj, sem_idx, sem_out):
    wid = lax.axis_index("s") * NUM_CORES + lax.axis_index("c")
    nhi = jnp.minimum(wid, NUM_HI)
    nlo = wid - nhi
    base = (nhi * CHUNKS_HI + nlo * CHUNKS_LO) * CHUNK
    num_chunks = jnp.where(wid < NUM_HI, CHUNKS_HI, CHUNKS_LO)

    @pl.when(lax.axis_index("s") == 0)
    def _():
        pltpu.async_copy(table_hbm, table_sh, sem_tab).wait()

    plsc.subcore_barrier()

    def xa_copy(n):
        s = lax.rem(n, 3)
        off = base + n * CHUNK
        return pltpu.make_async_copy(
            x_i_hbm.at[pl.ds(off, CHUNK), :], xa_v.at[s], sem_xa.at[s])

    def ga_start(n):
        s = lax.rem(n, 3)
        pltpu.async_copy(
            table_sh.at[idx_v.at[s]], xa_v.at[s], sem_ga.at[s], add=True)

    def ga_wait(n):
        s = lax.rem(n, 3)
        pltpu.make_async_copy(
            table_sh.at[idx_v.at[s]], xa_v.at[s], sem_ga.at[s]).wait()

    def xj_copy(n):
        s = lax.rem(n, 3)
        off = base + n * CHUNK
        return pltpu.make_async_copy(
            x_j_hbm.at[pl.ds(off, CHUNK), :], xj_v.at[s], sem_xj.at[s])

    def idx_copy(n):
        s = lax.rem(n, 3)
        off = base + n * CHUNK
        return pltpu.make_async_copy(
            et_hbm.at[pl.ds(off, CHUNK)], idx_v.at[s], sem_idx.at[s])

    def out_copy(n):
        s = lax.rem(n, 2)
        off = base + n * CHUNK
        return pltpu.make_async_copy(
            out_v.at[s], out_hbm.at[pl.ds(off, CHUNK)], sem_out.at[s])

    idx_copy(0).start()
    xa_copy(0).start()
    xj_copy(0).start()
    idx_copy(1).start()
    xa_copy(1).start()
    xj_copy(1).start()
    idx_copy(0).wait()
    xa_copy(0).wait()
    ga_start(0)

    lanes = lax.iota(jnp.int32, LANES)

    def chunk_body(n, _):
        s = lax.rem(n, 3)

        @pl.when(n + 1 < num_chunks)
        def _():
            idx_copy(n + 1).wait()
            xa_copy(n + 1).wait()
            ga_start(n + 1)

        @pl.when(n + 2 < num_chunks)
        def _():
            idx_copy(n + 2).start()
            xa_copy(n + 2).start()
            xj_copy(n + 2).start()

        ga_wait(n)
        xj_copy(n).wait()

        so = lax.rem(n, 2)

        @pl.when(n >= 2)
        def _():
            out_copy(n - 2).wait()

        def group_body(gr, _):
            e0 = gr * LANES
            out16 = jnp.zeros((LANES,), jnp.float32)
            for j in range(LANES):
                e = e0 + j
                acc = jnp.zeros((LANES,), jnp.float32)
                for k in range(SUBVECS):
                    sl = pl.ds(k * LANES, LANES)
                    acc = acc + jnp.abs(xa_v[s, e, sl] - xj_v[s, e, sl])
                sval = jnp.sum(acc)
                out16 = jnp.where(lanes == j, sval, out16)
            out_v[so, pl.ds(e0, LANES)] = out16
            return 0

        lax.fori_loop(0, GROUPS, group_body, 0)
        out_copy(n).start()
        return 0

    lax.fori_loop(0, num_chunks, chunk_body, 0)

    out_copy(num_chunks - 2).wait()
    out_copy(num_chunks - 1).wait()


@jax.jit
def kernel(x_i, x_j, edge_type, relation_embedding):
    mesh = plsc.VectorSubcoreMesh(core_axis_name="c", subcore_axis_name="s")
    run = pl.kernel(
        _sc_kernel,
        out_type=jax.ShapeDtypeStruct((N_EDGES,), jnp.float32),
        mesh=mesh,
        compiler_params=pltpu.CompilerParams(needs_layout_passes=False),
        scratch_types=[
            pltpu.VMEM_SHARED((NUM_RELS, D), jnp.float32),
            pltpu.VMEM((3, CHUNK, D), jnp.float32),
            pltpu.VMEM((3, CHUNK, D), jnp.float32),
            pltpu.VMEM((3, CHUNK), jnp.int32),
            pltpu.VMEM((2, CHUNK), jnp.float32),
            pltpu.VMEM((CHUNK * LANES,), jnp.float32),
            pltpu.SemaphoreType.DMA,
            pltpu.SemaphoreType.DMA((3,)),
            pltpu.SemaphoreType.DMA((3,)),
            pltpu.SemaphoreType.DMA((3,)),
            pltpu.SemaphoreType.DMA((3,)),
            pltpu.SemaphoreType.DMA((2,)),
        ],
    )
    return run(x_i, x_j, edge_type.astype(jnp.int32), relation_embedding)


# speedup vs baseline: 1.4931x; 1.4931x over previous
import functools

import jax
import jax.numpy as jnp
from jax import lax
from jax.experimental import pallas as pl
from jax.experimental.pallas import tpu as pltpu
from jax.experimental.pallas import tpu_sc as plsc

N_EDGES = 320000
NUM_RELS = 1000
D = 128
LANES = 16
SUBVECS = D // LANES
NUM_CORES = 2
NUM_SUBCORES = 16
NUM_WORKERS = NUM_CORES * NUM_SUBCORES
CHUNK = 128
TOTAL_CHUNKS = N_EDGES // CHUNK
CHUNKS_LO = TOTAL_CHUNKS // NUM_WORKERS
CHUNKS_HI = CHUNKS_LO + 1
NUM_HI = TOTAL_CHUNKS - CHUNKS_LO * NUM_WORKERS
GROUPS = CHUNK // LANES


def _sc_kernel(x_i_hbm, x_j_hbm, et_hbm, table_hbm, out_hbm,
               table_sh, xa_v, xj_v, idx_v, out_v, part_v,
               sem_tab, sem_xa, sem_ga, sem_xj, sem_idx, sem_out):
    wid = lax.axis_index("s") * NUM_CORES + lax.axis_index("c")
    nhi = jnp.minimum(wid, NUM_HI)
    nlo = wid - nhi
    base = (nhi * CHUNKS_HI + nlo * CHUNKS_LO) * CHUNK
    num_chunks = jnp.where(wid < NUM_HI, CHUNKS_HI, CHUNKS_LO)

    @pl.when(lax.axis_index("s") == 0)
    def _():
        pltpu.async_copy(table_hbm, table_sh, sem_tab).wait()

    plsc.subcore_barrier()

    def xa_copy(n):
        s = lax.rem(n, 3)
        off = base + n * CHUNK
        return pltpu.make_async_copy(
            x_i_hbm.at[pl.ds(off, CHUNK), :], xa_v.at[s], sem_xa.at[s])

    def ga_start(n):
        s = lax.rem(n, 3)
        pltpu.async_copy(
            table_sh.at[idx_v.at[s]], xa_v.at[s], sem_ga.at[s], add=True)

    def ga_wait(n):
        s = lax.rem(n, 3)
        pltpu.make_async_copy(
            table_sh.at[idx_v.at[s]], xa_v.at[s], sem_ga.at[s]).wait()

    def xj_copy(n):
        s = lax.rem(n, 3)
        off = base + n * CHUNK
        return pltpu.make_async_copy(
            x_j_hbm.at[pl.ds(off, CHUNK), :], xj_v.at[s], sem_xj.at[s])

    def idx_copy(n):
        s = lax.rem(n, 3)
        off = base + n * CHUNK
        return pltpu.make_async_copy(
            et_hbm.at[pl.ds(off, CHUNK)], idx_v.at[s], sem_idx.at[s])

    def out_copy(n):
        s = lax.rem(n, 2)
        off = base + n * CHUNK
        return pltpu.make_async_copy(
            out_v.at[s], out_hbm.at[pl.ds(off, CHUNK)], sem_out.at[s])

    idx_copy(0).start()
    xa_copy(0).start()
    xj_copy(0).start()
    idx_copy(1).start()
    xa_copy(1).start()
    xj_copy(1).start()
    idx_copy(0).wait()
    xa_copy(0).wait()
    ga_start(0)

    lanes = lax.iota(jnp.int32, LANES)

    def chunk_body(n, _):
        s = lax.rem(n, 3)

        @pl.when(n + 1 < num_chunks)
        def _():
            idx_copy(n + 1).wait()
            xa_copy(n + 1).wait()
            ga_start(n + 1)

        @pl.when(n + 2 < num_chunks)
        def _():
            idx_copy(n + 2).start()
            xa_copy(n + 2).start()
            xj_copy(n + 2).start()

        ga_wait(n)
        xj_copy(n).wait()

        so = lax.rem(n, 2)

        @pl.when(n >= 2)
        def _():
            out_copy(n - 2).wait()

        def edge_body(e, _):
            acc = jnp.zeros((LANES,), jnp.float32)
            for k in range(SUBVECS):
                sl = pl.ds(k * LANES, LANES)
                acc = acc + jnp.abs(xa_v[s, e, sl] - xj_v[s, e, sl])
            part_v[pl.ds(e * LANES, LANES)] = acc
            return 0

        lax.fori_loop(0, CHUNK, edge_body, 0, unroll=8)

        def group_body(gr, _):
            rowbase = gr * (LANES * LANES) + lanes * LANES
            tot = jnp.zeros((LANES,), jnp.float32)
            for k in range(LANES):
                tot = tot + plsc.load_gather(part_v, [rowbase + k])
            out_v[so, pl.ds(gr * LANES, LANES)] = tot
            return 0

        lax.fori_loop(0, GROUPS, group_body, 0)
        out_copy(n).start()
        return 0

    lax.fori_loop(0, num_chunks, chunk_body, 0)

    out_copy(num_chunks - 2).wait()
    out_copy(num_chunks - 1).wait()


@jax.jit
def kernel(x_i, x_j, edge_type, relation_embedding):
    mesh = plsc.VectorSubcoreMesh(core_axis_name="c", subcore_axis_name="s")
    run = pl.kernel(
        _sc_kernel,
        out_type=jax.ShapeDtypeStruct((N_EDGES,), jnp.float32),
        mesh=mesh,
        compiler_params=pltpu.CompilerParams(needs_layout_passes=False),
        scratch_types=[
            pltpu.VMEM_SHARED((NUM_RELS, D), jnp.float32),
            pltpu.VMEM((3, CHUNK, D), jnp.float32),
            pltpu.VMEM((3, CHUNK, D), jnp.float32),
            pltpu.VMEM((3, CHUNK), jnp.int32),
            pltpu.VMEM((2, CHUNK), jnp.float32),
            pltpu.VMEM((CHUNK * LANES,), jnp.float32),
            pltpu.SemaphoreType.DMA,
            pltpu.SemaphoreType.DMA((3,)),
            pltpu.SemaphoreType.DMA((3,)),
            pltpu.SemaphoreType.DMA((3,)),
            pltpu.SemaphoreType.DMA((3,)),
            pltpu.SemaphoreType.DMA((2,)),
        ],
    )
    return run(x_i, x_j, edge_type.astype(jnp.int32), relation_embedding)
